# baseline (speedup 1.0000x reference)
import functools
import numpy as np
import jax
import jax.numpy as jnp
from jax import lax
from jax.experimental import pallas as pl
from jax.experimental.pallas import tpu as pltpu
from jax.experimental.pallas import tpu_sc as plsc

_B, _L, _C = 4096, 200, 21
_NS = 50
_SPT = 16
_NW = 32

_tab = np.zeros((_C, 32), np.float32)
for _v in range(1, 22):
    _tab[_v - 1, _v] = 1.0
for _c, _v in ((2, 22), (11, 22), (3, 23), (13, 23), (7, 24), (9, 24)):
    _tab[_c, _v] = 0.5
_TABLE = _tab.reshape(-1)

_mesh = plsc.VectorSubcoreMesh(core_axis_name="c", subcore_axis_name="s")


@functools.partial(
    pl.kernel,
    out_type=jax.ShapeDtypeStruct((_C * _NS, _SPT, 1024), jnp.float32),
    mesh=_mesh,
    compiler_params=pltpu.CompilerParams(needs_layout_passes=False),
    scratch_types=[
        pltpu.VMEM((_SPT, 1024), jnp.int32),
        pltpu.VMEM((_SPT, 1024), jnp.float32),
        pltpu.VMEM((_C * 32,), jnp.float32),
    ],
)
def _sc_lookup(tab_hbm, seq_hbm, out_hbm, tok_v, stage_v, tab_v):
    wid = lax.axis_index("s") * 2 + lax.axis_index("c")
    pltpu.sync_copy(tab_hbm, tab_v)

    def process_span(s):
        pltpu.sync_copy(seq_hbm.at[s], tok_v)

        def c_body(c, carry):
            def t_body(t, carry):
                def j_body(j, carry):
                    v16 = tok_v[t, pl.ds(j * 16, 16)]
                    vals = plsc.load_gather(tab_v, [v16 + c * 32])
                    stage_v[t, pl.ds(j * 16, 16)] = vals
                    return carry

                return lax.fori_loop(0, 64, j_body, carry)

            carry = lax.fori_loop(0, _SPT, t_body, carry)
            pltpu.sync_copy(stage_v, out_hbm.at[c * _NS + s])
            return carry

        lax.fori_loop(0, _C, c_body, 0)

    process_span(wid)

    @pl.when(wid < _NS - _NW)
    def _():
        process_span(wid + _NW)


def kernel(sequence):
    seq_t = sequence.T
    seq_spans = (
        seq_t.reshape(25, 8, 32, 128)
        .transpose(0, 2, 1, 3)
        .reshape(_NS, _SPT, 1024)
    )
    out_spans = _sc_lookup(jnp.asarray(_TABLE), seq_spans)
    out_t = (
        out_spans.reshape(_C, 25, 32, 8, 128)
        .transpose(0, 1, 3, 2, 4)
        .reshape(_C, _L, _B)
    )
    return out_t.transpose(2, 1, 0)


# speedup vs baseline: 1.0257x; 1.0257x over previous
import functools
import numpy as np
import jax
import jax.numpy as jnp
from jax import lax
from jax.experimental import pallas as pl
from jax.experimental.pallas import tpu as pltpu
from jax.experimental.pallas import tpu_sc as plsc

_B, _L, _C = 4096, 200, 21
_NS = 100
_SPT = 8
_NW = 32

_tab = np.zeros((_C, 32), np.float32)
for _v in range(1, 22):
    _tab[_v - 1, _v] = 1.0
for _c, _v in ((2, 22), (11, 22), (3, 23), (13, 23), (7, 24), (9, 24)):
    _tab[_c, _v] = 0.5
_TABLE = _tab.reshape(-1)

_mesh = plsc.VectorSubcoreMesh(core_axis_name="c", subcore_axis_name="s")


@functools.partial(
    pl.kernel,
    out_type=jax.ShapeDtypeStruct((_C * _NS, _SPT, 1024), jnp.float32),
    mesh=_mesh,
    compiler_params=pltpu.CompilerParams(needs_layout_passes=False),
    scratch_types=[
        pltpu.VMEM((_SPT, 1024), jnp.int32),
        pltpu.VMEM((_SPT, 1024), jnp.float32),
        pltpu.VMEM((_C * 32,), jnp.float32),
    ],
)
def _sc_lookup(tab_hbm, seq_hbm, out_hbm, tok_v, stage_v, tab_v):
    wid = lax.axis_index("s") * 2 + lax.axis_index("c")
    pltpu.sync_copy(tab_hbm, tab_v)

    def process_span(s):
        pltpu.sync_copy(seq_hbm.at[s], tok_v)

        def c_body(c, carry):
            def t_body(t, carry):
                base = c * 32
                for j in range(64):
                    v16 = tok_v[t, pl.ds(j * 16, 16)]
                    vals = plsc.load_gather(tab_v, [v16 + base])
                    stage_v[t, pl.ds(j * 16, 16)] = vals
                return carry

            carry = lax.fori_loop(0, _SPT, t_body, carry)
            pltpu.sync_copy(stage_v, out_hbm.at[c * _NS + s])
            return carry

        lax.fori_loop(0, _C, c_body, 0)

    def span_loop(r, carry):
        s = wid + r * _NW

        @pl.when(s < _NS)
        def _():
            process_span(s)

        return carry

    lax.fori_loop(0, (_NS + _NW - 1) // _NW, span_loop, 0)


def kernel(sequence):
    seq_t = sequence.T
    seq_spans = (
        seq_t.reshape(25, 8, 32, 128)
        .transpose(0, 2, 1, 3)
        .reshape(_NS, _SPT, 1024)
    )
    out_spans = _sc_lookup(jnp.asarray(_TABLE), seq_spans)
    out_t = (
        out_spans.reshape(_C, 25, 32, 8, 128)
        .transpose(0, 1, 3, 2, 4)
        .reshape(_C, _L, _B)
    )
    return out_t.transpose(2, 1, 0)


# speedup vs baseline: 11.8717x; 11.5745x over previous
import jax
import jax.numpy as jnp
from jax import lax
from jax.experimental import pallas as pl

_B, _L, _C = 4096, 200, 21
_BB = 512

_SPECIAL = {2: 22, 11: 22, 3: 23, 13: 23, 7: 24, 9: 24}


def _body(seq_ref, out_ref):
    s = seq_ref[...]
    half = {
        22: jnp.where(s == 22, 0.5, 0.0),
        23: jnp.where(s == 23, 0.5, 0.0),
        24: jnp.where(s == 24, 0.5, 0.0),
    }
    for c in range(_C):
        v = jnp.where(s == c + 1, 1.0, 0.0)
        if c in _SPECIAL:
            v = v + half[_SPECIAL[c]]
        out_ref[c, :, :] = v


def kernel(sequence):
    seq_t = sequence.T
    out_t = pl.pallas_call(
        _body,
        grid=(_B // _BB,),
        in_specs=[pl.BlockSpec((_L, _BB), lambda i: (0, i))],
        out_specs=pl.BlockSpec((_C, _L, _BB), lambda i: (0, 0, i)),
        out_shape=jax.ShapeDtypeStruct((_C, _L, _B), jnp.float32),
    )(seq_t)
    return out_t.transpose(2, 1, 0)
